# baseline (speedup 1.0000x reference)
import functools
import math

import jax
import jax.numpy as jnp
from jax import lax
from jax.experimental import pallas as pl
from jax.experimental.pallas import tpu as pltpu
from jax.experimental.pallas import tpu_sc as plsc

_B, _H, _S = 2, 12, 2048
_R = 1024
_NR = (_H * _S) // _R


def _rate(i=8, num_hidden_layers=12, token_keep_rate=0.5):
    layers_before = max(3, math.ceil(0.15 * num_hidden_layers))
    layers_with = num_hidden_layers - layers_before
    if i < layers_before:
        return 1.0
    m = (token_keep_rate - 1.0) / layers_with
    return max(0.01, m * (i - layers_before + 1) + 1.0)


def _reduce_body(x_ref, o_ref, acc_ref):
    b = pl.program_id(0)
    r = pl.program_id(1)

    @pl.when(r == 0)
    def _init():
        acc_ref[b] = jnp.zeros((8, _S), jnp.float32)

    acc = acc_ref[b]
    for i in range(_R // 8):
        acc = acc + x_ref[0, i * 8 : (i + 1) * 8, :]
    acc_ref[b] = acc

    @pl.when(r == _NR - 1)
    def _fold():
        a = acc_ref[b]
        t1 = a[0:4] + a[4:8]
        t2 = t1[0:2] + t1[2:4]
        o_ref[b] = t2[0:1] + t2[1:2]


def _sc_topk(scores, keep16):
    mesh = plsc.VectorSubcoreMesh(core_axis_name="c", subcore_axis_name="s")

    @functools.partial(
        pl.kernel,
        mesh=mesh,
        compiler_params=pltpu.CompilerParams(needs_layout_passes=False),
        out_type=jax.ShapeDtypeStruct((_B, 1, _S), jnp.float32),
        scratch_types=[
            pltpu.VMEM((_S,), jnp.int32),
            pltpu.VMEM((16,), jnp.int32),
            pltpu.VMEM((_S,), jnp.float32),
        ],
    )
    def sc_kernel(scores_hbm, keep_hbm, out_hbm, u_v, k_v, o_v):
        wid = lax.axis_index("s") * 2 + lax.axis_index("c")
        b = jnp.minimum(wid, _B - 1)
        pltpu.sync_copy(scores_hbm.at[b, 0], u_v)
        pltpu.sync_copy(keep_hbm.at[b], k_v)
        lane = lax.broadcasted_iota(jnp.int32, (16,), 0)
        kb = k_v[...]
        zero = jnp.zeros((16,), jnp.int32)

        def _count(pred):
            def body(i, acc):
                ui = u_v[pl.ds(i * 16, 16)]
                iv = lane + i * 16
                return acc + plsc.all_reduce_population_count(pred(ui, iv))

            return lax.fori_loop(0, _S // 16, body, zero)

        def vbit(j, t):
            cand = t | lax.shift_left(jnp.full((16,), 1, jnp.int32), 30 - j)
            cnt = _count(lambda ui, iv: ui >= cand)
            return jnp.where(cnt >= kb, cand, t)

        t = lax.fori_loop(0, 31, vbit, zero)
        c_gt = _count(lambda ui, iv: ui > t)
        need = kb - c_gt
        def mbit(j, m):
            cand = m | lax.shift_left(jnp.full((16,), 1, jnp.int32), 11 - j)
            cnt = _count(lambda ui, iv: (ui == t) & (iv <= cand))
            return jnp.where(cnt < need, cand, m)

        m0 = lax.fori_loop(0, 12, mbit, zero)
        cnt0 = _count(lambda ui, iv: (ui == t) & (iv <= m0))
        mstar = jnp.where(
            cnt0 < need,
            m0 + 1,
            jnp.where(need > 0, zero, jnp.full((16,), -1, jnp.int32)),
        )

        def wbody(i, carry):
            ui = u_v[pl.ds(i * 16, 16)]
            iv = lane + i * 16
            keep = (ui > t) | ((ui == t) & (iv <= mstar))
            keep = keep & (kb > 0)
            o_v[pl.ds(i * 16, 16)] = jnp.where(
                keep, jnp.float32(0.0), jnp.float32(-10000.0)
            )
            return carry

        lax.fori_loop(0, _S // 16, wbody, jnp.int32(0))

        @pl.when(wid < _B)
        def _write():
            pltpu.sync_copy(o_v, out_hbm.at[b, 0])

    return sc_kernel(scores, keep16)


def kernel(attention_mask, attention_probs, sentence_lengths):
    rate = _rate()
    if rate == 1.0:
        return attention_mask
    keep_tokens = jnp.round(sentence_lengths.astype(jnp.float32) * rate).astype(
        jnp.int32
    )
    keep16 = jnp.tile(keep_tokens[:, None], (1, 16))
    B, H, S, _ = attention_probs.shape
    probs3 = attention_probs.reshape(B, H * S, S)
    scores = pl.pallas_call(
        _reduce_body,
        grid=(B, _NR),
        in_specs=[pl.BlockSpec((1, _R, S), lambda b, r: (b, r, 0))],
        out_specs=pl.BlockSpec((B, 1, S), lambda b, r: (0, 0, 0)),
        out_shape=jax.ShapeDtypeStruct((B, 1, S), jnp.float32),
        scratch_shapes=[pltpu.VMEM((B, 8, S), jnp.float32)],
    )(probs3)
    out = _sc_topk(lax.bitcast_convert_type(scores, jnp.int32), keep16)
    return out.reshape(B, 1, 1, S)
